# baseline (speedup 1.0000x reference)
import functools

import jax
import jax.numpy as jnp
from jax import lax
from jax.experimental import pallas as pl
from jax.experimental.pallas import tpu as pltpu
from jax.experimental.pallas import tpu_sc as plsc

CONF_THRESH = 0.55
IOU_THRESH = 0.5
MAX_OUT = 100
N_BOXES = 20000

_NT = 32
_PER = 640
_NP = _NT * _PER
_CHUNKS = _PER // 16
_CAP = _PER + 16
_TAG_BASE = 2000003.0

_mesh = plsc.VectorSubcoreMesh(core_axis_name="c", subcore_axis_name="s")


def _f(v):
    return jnp.full((16,), v, jnp.float32)


def _i(v):
    return jnp.full((16,), v, jnp.int32)


@functools.partial(
    pl.kernel,
    out_type=jax.ShapeDtypeStruct((MAX_OUT, 16), jnp.float32),
    mesh=_mesh,
    compiler_params=pltpu.CompilerParams(needs_layout_passes=False),
    scratch_types=[
        pltpu.VMEM((_PER,), jnp.float32),
        pltpu.VMEM((_PER,), jnp.float32),
        pltpu.VMEM((_PER,), jnp.float32),
        pltpu.VMEM((_PER,), jnp.float32),
        pltpu.VMEM((_PER,), jnp.float32),
        pltpu.VMEM((_CAP,), jnp.float32),
        pltpu.VMEM((_CAP,), jnp.float32),
        pltpu.VMEM((_CAP,), jnp.float32),
        pltpu.VMEM((_CAP,), jnp.float32),
        pltpu.VMEM((_CAP,), jnp.float32),
        pltpu.VMEM((_CAP,), jnp.int32),
        pltpu.VMEM((16,), jnp.float32),
        pltpu.VMEM((_NT, 16), jnp.float32),
        pltpu.VMEM((MAX_OUT, 16), jnp.float32),
        pltpu.HBM((2, _NT, 16), jnp.float32),
    ],
)
def _nms_sc(x1h, y1h, x2h, y2h, sh, out_h,
            x1v, y1v, x2v, y2v, sv,
            c1v, c2v, c3v, c4v, csv, qiv,
            pubv, rdv, outv, shared):
    c = lax.axis_index("c")
    t = lax.axis_index("s")
    w = c * 16 + t
    base = w * _PER

    pltpu.sync_copy(x1h.at[pl.ds(base, _PER)], x1v)
    pltpu.sync_copy(y1h.at[pl.ds(base, _PER)], y1v)
    pltpu.sync_copy(x2h.at[pl.ds(base, _PER)], x2v)
    pltpu.sync_copy(y2h.at[pl.ds(base, _PER)], y2v)
    pltpu.sync_copy(sh.at[pl.ds(base, _PER)], sv)

    lane = lax.iota(jnp.int32, 16)

    def prefill(ck, carry):
        off = ck * 16
        c1v[pl.ds(off, 16)] = _f(0.0)
        c2v[pl.ds(off, 16)] = _f(0.0)
        c3v[pl.ds(off, 16)] = _f(0.0)
        c4v[pl.ds(off, 16)] = _f(0.0)
        csv[pl.ds(off, 16)] = _f(-1.0)
        qiv[pl.ds(off, 16)] = _i(0)
        return carry

    lax.fori_loop(0, _CAP // 16, prefill, 0)

    def compact(ck, cnt):
        off = ck * 16
        sc = sv[pl.ds(off, 16)]
        m = sc > CONF_THRESH
        plsc.store_compressed(c1v.at[pl.ds(cnt, 16)], x1v[pl.ds(off, 16)], mask=m)
        plsc.store_compressed(c2v.at[pl.ds(cnt, 16)], y1v[pl.ds(off, 16)], mask=m)
        plsc.store_compressed(c3v.at[pl.ds(cnt, 16)], x2v[pl.ds(off, 16)], mask=m)
        plsc.store_compressed(c4v.at[pl.ds(cnt, 16)], y2v[pl.ds(off, 16)], mask=m)
        plsc.store_compressed(csv.at[pl.ds(cnt, 16)], sc, mask=m)
        plsc.store_compressed(qiv.at[pl.ds(cnt, 16)], _i(base + off) + lane, mask=m)
        return cnt + jnp.max(plsc.all_reduce_population_count(m))

    cnt = lax.fori_loop(0, _CHUNKS, compact, jnp.int32(0))
    nblk = (cnt + 63) // 64

    def fused_pass(wx1, wy1, wx2, wy2):
        area_a = (wx2 - wx1) * (wy2 - wy1)

        def blk_body(blk, carry):
            bv, bi = carry
            for jj in range(4):
                off = blk * 64 + jj * 16
                xa = c1v[pl.ds(off, 16)]
                ya = c2v[pl.ds(off, 16)]
                xb = c3v[pl.ds(off, 16)]
                yb = c4v[pl.ds(off, 16)]
                sc = csv[pl.ds(off, 16)]
                ix1 = jnp.maximum(_f(wx1), xa)
                iy1 = jnp.maximum(_f(wy1), ya)
                ix2 = jnp.minimum(_f(wx2), xb)
                iy2 = jnp.minimum(_f(wy2), yb)
                inter = (jnp.maximum(ix2 - ix1, _f(0.0))
                         * jnp.maximum(iy2 - iy1, _f(0.0)))
                area_b = (xb - xa) * (yb - ya)
                iou = inter / (_f(area_a) + area_b - inter + _f(1e-9))
                sc = jnp.where(iou > IOU_THRESH, _f(-1.0), sc)
                csv[pl.ds(off, 16)] = sc
                cond = sc > bv
                bv = jnp.where(cond, sc, bv)
                bi = jnp.where(cond, _i(off) + lane, bi)
            return bv, bi

        bv, bi = lax.fori_loop(0, nblk, blk_body,
                               (_f(-2.0), _i(2 ** 30)))
        lmax = jnp.max(bv)
        lidx = jnp.min(jnp.where(bv == _f(lmax), bi, _i(2 ** 30)))
        return lmax, lidx

    lmax0, lidx0 = fused_pass(jnp.float32(-10.0), jnp.float32(-10.0),
                              jnp.float32(-9.0), jnp.float32(-9.0))

    def pick(i, carry):
        lmax, lidx = carry
        loc = _i(jnp.minimum(lidx, _CAP - 1))
        gid = plsc.load_gather(qiv, [loc])
        cx1 = plsc.load_gather(c1v, [loc])
        cy1 = plsc.load_gather(c2v, [loc])
        cx2 = plsc.load_gather(c3v, [loc])
        cy2 = plsc.load_gather(c4v, [loc])
        tag = _TAG_BASE + i.astype(jnp.float32)
        pub = jnp.where(lane == _i(0), _f(lmax), _f(0.0))
        pub = jnp.where(lane == _i(1), gid.astype(jnp.float32), pub)
        pub = jnp.where(lane == _i(2), cx1, pub)
        pub = jnp.where(lane == _i(3), cy1, pub)
        pub = jnp.where(lane == _i(4), cx2, pub)
        pub = jnp.where(lane == _i(5), cy2, pub)
        pub = jnp.where(lane == _i(6), _f(tag), pub)
        pubv[...] = pub
        par = lax.rem(i, 2)
        pltpu.sync_copy(pubv, shared.at[par, w])
        plsc.subcore_barrier()

        def colh(j, h):
            return plsc.load_gather(rdv, [lane + _i(16 * h), _i(j)])

        def read_once(_):
            pltpu.sync_copy(shared.at[par], rdv)
            return jnp.all((colh(6, 0) == _f(tag)) & (colh(6, 1) == _f(tag)))

        ok_read = read_once(False)
        lax.while_loop(lambda o: ~o, read_once, ok_read)

        s0, s1 = colh(0, 0), colh(0, 1)
        i0, i1 = colh(1, 0), colh(1, 1)
        m = jnp.maximum(jnp.max(s0), jnp.max(s1))
        widx = jnp.minimum(
            jnp.min(jnp.where(s0 == _f(m), i0, _f(3.0e7))),
            jnp.min(jnp.where(s1 == _f(m), i1, _f(3.0e7))))
        wsel0 = (s0 == _f(m)) & (i0 == _f(widx))
        wsel1 = (s1 == _f(m)) & (i1 == _f(widx))

        def pickf(j):
            return jnp.maximum(
                jnp.max(jnp.where(wsel0, colh(j, 0), _f(-3.0e7))),
                jnp.max(jnp.where(wsel1, colh(j, 1), _f(-3.0e7))))

        wx1 = pickf(2)
        wy1 = pickf(3)
        wx2 = pickf(4)
        wy2 = pickf(5)
        ok = m > 0.0

        @pl.when(w == 0)
        def _emit():
            row = jnp.where(lane == _i(0), _f(wx1), _f(0.0))
            row = jnp.where(lane == _i(1), _f(wy1), row)
            row = jnp.where(lane == _i(2), _f(wx2), row)
            row = jnp.where(lane == _i(3), _f(wy2), row)
            row = jnp.where(lane == _i(4), _f(m), row)
            row = jnp.where(jnp.full((16,), ok, jnp.bool_), row, _f(0.0))
            plsc.store_scatter(outv, [_i(i), lane], row)

        return fused_pass(wx1, wy1, wx2, wy2)

    lax.fori_loop(0, MAX_OUT, pick, (lmax0, lidx0))

    @pl.when(w == 0)
    def _flush():
        pltpu.sync_copy(outv, out_h)


@jax.jit
def kernel(boxes, scores):
    padn = _NP - N_BOXES
    bt = jnp.pad(boxes, ((0, padn), (0, 0))).T
    sp = jnp.pad(scores, (0, padn), constant_values=-1.0)
    out = _nms_sc(bt[0], bt[1], bt[2], bt[3], sp)
    return out[:, :5]


# speedup vs baseline: 1.5388x; 1.5388x over previous
import functools

import jax
import jax.numpy as jnp
from jax import lax
from jax.experimental import pallas as pl
from jax.experimental.pallas import tpu as pltpu
from jax.experimental.pallas import tpu_sc as plsc

CONF_THRESH = 0.55
IOU_THRESH = 0.5
MAX_OUT = 100
N_BOXES = 20000

_NT = 16
_PER = 1280
_NP = _NT * _PER
_CHUNKS = _PER // 16
_CAP = _PER + 16

_mesh = plsc.VectorSubcoreMesh(core_axis_name="c", subcore_axis_name="s")


def _f(v):
    return jnp.full((16,), v, jnp.float32)


def _i(v):
    return jnp.full((16,), v, jnp.int32)


@functools.partial(
    pl.kernel,
    out_type=jax.ShapeDtypeStruct((MAX_OUT, 16), jnp.float32),
    mesh=_mesh,
    compiler_params=pltpu.CompilerParams(needs_layout_passes=False),
    scratch_types=[
        pltpu.VMEM((_PER,), jnp.float32),
        pltpu.VMEM((_PER,), jnp.float32),
        pltpu.VMEM((_PER,), jnp.float32),
        pltpu.VMEM((_PER,), jnp.float32),
        pltpu.VMEM((_PER,), jnp.float32),
        pltpu.VMEM((_CAP,), jnp.float32),
        pltpu.VMEM((_CAP,), jnp.float32),
        pltpu.VMEM((_CAP,), jnp.float32),
        pltpu.VMEM((_CAP,), jnp.float32),
        pltpu.VMEM((_CAP,), jnp.float32),
        pltpu.VMEM((_CAP,), jnp.int32),
        pltpu.VMEM((16,), jnp.float32),
        pltpu.VMEM((_NT, 16), jnp.float32),
        pltpu.VMEM((MAX_OUT, 16), jnp.float32),
        pltpu.HBM((2, _NT, 16), jnp.float32),
    ],
)
def _nms_sc(x1h, y1h, x2h, y2h, sh, out_h,
            x1v, y1v, x2v, y2v, sv,
            c1v, c2v, c3v, c4v, csv, qiv,
            pubv, rdv, outv, shared):
    c = lax.axis_index("c")
    t = lax.axis_index("s")

    @pl.when(c == 0)
    def _body():
        base = t * _PER
        pltpu.sync_copy(x1h.at[pl.ds(base, _PER)], x1v)
        pltpu.sync_copy(y1h.at[pl.ds(base, _PER)], y1v)
        pltpu.sync_copy(x2h.at[pl.ds(base, _PER)], x2v)
        pltpu.sync_copy(y2h.at[pl.ds(base, _PER)], y2v)
        pltpu.sync_copy(sh.at[pl.ds(base, _PER)], sv)

        lane = lax.iota(jnp.int32, 16)

        def prefill(ck, carry):
            off = ck * 16
            c1v[pl.ds(off, 16)] = _f(0.0)
            c2v[pl.ds(off, 16)] = _f(0.0)
            c3v[pl.ds(off, 16)] = _f(0.0)
            c4v[pl.ds(off, 16)] = _f(0.0)
            csv[pl.ds(off, 16)] = _f(-1.0)
            qiv[pl.ds(off, 16)] = _i(0)
            return carry

        lax.fori_loop(0, _CAP // 16, prefill, 0)

        def compact(ck, cnt):
            off = ck * 16
            sc = sv[pl.ds(off, 16)]
            m = sc > CONF_THRESH
            plsc.store_compressed(c1v.at[pl.ds(cnt, 16)], x1v[pl.ds(off, 16)], mask=m)
            plsc.store_compressed(c2v.at[pl.ds(cnt, 16)], y1v[pl.ds(off, 16)], mask=m)
            plsc.store_compressed(c3v.at[pl.ds(cnt, 16)], x2v[pl.ds(off, 16)], mask=m)
            plsc.store_compressed(c4v.at[pl.ds(cnt, 16)], y2v[pl.ds(off, 16)], mask=m)
            plsc.store_compressed(csv.at[pl.ds(cnt, 16)], sc, mask=m)
            plsc.store_compressed(qiv.at[pl.ds(cnt, 16)], _i(base + off) + lane, mask=m)
            return cnt + jnp.max(plsc.all_reduce_population_count(m))

        cnt = lax.fori_loop(0, _CHUNKS, compact, jnp.int32(0))
        nblk = (cnt + 63) // 64

        def fused_pass(wx1, wy1, wx2, wy2):
            area_a = (wx2 - wx1) * (wy2 - wy1)

            def blk_body(blk, carry):
                bv, bi = carry
                for jj in range(4):
                    off = blk * 64 + jj * 16
                    xa = c1v[pl.ds(off, 16)]
                    ya = c2v[pl.ds(off, 16)]
                    xb = c3v[pl.ds(off, 16)]
                    yb = c4v[pl.ds(off, 16)]
                    sc = csv[pl.ds(off, 16)]
                    ix1 = jnp.maximum(_f(wx1), xa)
                    iy1 = jnp.maximum(_f(wy1), ya)
                    ix2 = jnp.minimum(_f(wx2), xb)
                    iy2 = jnp.minimum(_f(wy2), yb)
                    inter = (jnp.maximum(ix2 - ix1, _f(0.0))
                             * jnp.maximum(iy2 - iy1, _f(0.0)))
                    area_b = (xb - xa) * (yb - ya)
                    iou = inter / (_f(area_a) + area_b - inter + _f(1e-9))
                    sc = jnp.where(iou > IOU_THRESH, _f(-1.0), sc)
                    csv[pl.ds(off, 16)] = sc
                    cond = sc > bv
                    bv = jnp.where(cond, sc, bv)
                    bi = jnp.where(cond, _i(off) + lane, bi)
                return bv, bi

            bv, bi = plsc.parallel_loop(
                0, nblk, step=1, carry=(_f(-2.0), _i(2 ** 30)))(blk_body)
            lmax = jnp.max(bv)
            lidx = jnp.min(jnp.where(bv == _f(lmax), bi, _i(2 ** 30)))
            return lmax, lidx

        lmax0, lidx0 = fused_pass(jnp.float32(-10.0), jnp.float32(-10.0),
                                  jnp.float32(-9.0), jnp.float32(-9.0))

        def pick(i, carry):
            lmax, lidx = carry
            loc = _i(jnp.minimum(lidx, _CAP - 1))
            gid = plsc.load_gather(qiv, [loc])
            cx1 = plsc.load_gather(c1v, [loc])
            cy1 = plsc.load_gather(c2v, [loc])
            cx2 = plsc.load_gather(c3v, [loc])
            cy2 = plsc.load_gather(c4v, [loc])
            pub = jnp.where(lane == _i(0), _f(lmax), _f(0.0))
            pub = jnp.where(lane == _i(1), gid.astype(jnp.float32), pub)
            pub = jnp.where(lane == _i(2), cx1, pub)
            pub = jnp.where(lane == _i(3), cy1, pub)
            pub = jnp.where(lane == _i(4), cx2, pub)
            pub = jnp.where(lane == _i(5), cy2, pub)
            pubv[...] = pub
            par = lax.rem(i, 2)
            pltpu.sync_copy(pubv, shared.at[par, t])
            plsc.subcore_barrier()
            pltpu.sync_copy(shared.at[par], rdv)

            def col(j):
                return plsc.load_gather(rdv, [lane, _i(j)])

            scores = col(0)
            idxs = col(1)
            m = jnp.max(scores)
            widx = jnp.min(jnp.where(scores == _f(m), idxs, _f(3.0e7)))
            wsel = (scores == _f(m)) & (idxs == _f(widx))
            wx1 = jnp.max(jnp.where(wsel, col(2), _f(-3.0e7)))
            wy1 = jnp.max(jnp.where(wsel, col(3), _f(-3.0e7)))
            wx2 = jnp.max(jnp.where(wsel, col(4), _f(-3.0e7)))
            wy2 = jnp.max(jnp.where(wsel, col(5), _f(-3.0e7)))
            ok = m > 0.0

            @pl.when(t == 0)
            def _emit():
                row = jnp.where(lane == _i(0), _f(wx1), _f(0.0))
                row = jnp.where(lane == _i(1), _f(wy1), row)
                row = jnp.where(lane == _i(2), _f(wx2), row)
                row = jnp.where(lane == _i(3), _f(wy2), row)
                row = jnp.where(lane == _i(4), _f(m), row)
                row = jnp.where(jnp.full((16,), ok, jnp.bool_), row, _f(0.0))
                plsc.store_scatter(outv, [_i(i), lane], row)

            return fused_pass(wx1, wy1, wx2, wy2)

        lax.fori_loop(0, MAX_OUT, pick, (lmax0, lidx0))

        @pl.when(t == 0)
        def _flush():
            pltpu.sync_copy(outv, out_h)


@jax.jit
def kernel(boxes, scores):
    padn = _NP - N_BOXES
    bt = jnp.pad(boxes, ((0, padn), (0, 0))).T
    sp = jnp.pad(scores, (0, padn), constant_values=-1.0)
    out = _nms_sc(bt[0], bt[1], bt[2], bt[3], sp)
    return out[:, :5]


# speedup vs baseline: 1.5507x; 1.0077x over previous
import functools

import jax
import jax.numpy as jnp
from jax import lax
from jax.experimental import pallas as pl
from jax.experimental.pallas import tpu as pltpu
from jax.experimental.pallas import tpu_sc as plsc

CONF_THRESH = 0.55
IOU_THRESH = 0.5
MAX_OUT = 100
N_BOXES = 20000

_NT = 16
_PER = 1280
_NP = _NT * _PER
_CHUNKS = _PER // 16
_CAP = _PER + 16

_mesh = plsc.VectorSubcoreMesh(core_axis_name="c", subcore_axis_name="s")


def _f(v):
    return jnp.full((16,), v, jnp.float32)


def _i(v):
    return jnp.full((16,), v, jnp.int32)


@functools.partial(
    pl.kernel,
    out_type=jax.ShapeDtypeStruct((MAX_OUT, 16), jnp.float32),
    mesh=_mesh,
    compiler_params=pltpu.CompilerParams(needs_layout_passes=False),
    scratch_types=[
        pltpu.VMEM((_PER,), jnp.float32),
        pltpu.VMEM((_PER,), jnp.float32),
        pltpu.VMEM((_PER,), jnp.float32),
        pltpu.VMEM((_PER,), jnp.float32),
        pltpu.VMEM((_PER,), jnp.float32),
        pltpu.VMEM((_CAP,), jnp.float32),
        pltpu.VMEM((_CAP,), jnp.float32),
        pltpu.VMEM((_CAP,), jnp.float32),
        pltpu.VMEM((_CAP,), jnp.float32),
        pltpu.VMEM((_CAP,), jnp.float32),
        pltpu.VMEM((_CAP,), jnp.int32),
        pltpu.VMEM((16,), jnp.float32),
        pltpu.VMEM((_NT, 16), jnp.float32),
        pltpu.VMEM((MAX_OUT, 16), jnp.float32),
        pltpu.HBM((2, _NT, 16), jnp.float32),
    ],
)
def _nms_sc(x1h, y1h, x2h, y2h, sh, out_h,
            x1v, y1v, x2v, y2v, sv,
            c1v, c2v, c3v, c4v, csv, qiv,
            pubv, rdv, outv, shared):
    c = lax.axis_index("c")
    t = lax.axis_index("s")

    @pl.when(c == 0)
    def _body():
        base = t * _PER
        pltpu.sync_copy(x1h.at[pl.ds(base, _PER)], x1v)
        pltpu.sync_copy(y1h.at[pl.ds(base, _PER)], y1v)
        pltpu.sync_copy(x2h.at[pl.ds(base, _PER)], x2v)
        pltpu.sync_copy(y2h.at[pl.ds(base, _PER)], y2v)
        pltpu.sync_copy(sh.at[pl.ds(base, _PER)], sv)

        lane = lax.iota(jnp.int32, 16)

        def prefill(ck, carry):
            off = ck * 16
            c1v[pl.ds(off, 16)] = _f(0.0)
            c2v[pl.ds(off, 16)] = _f(0.0)
            c3v[pl.ds(off, 16)] = _f(0.0)
            c4v[pl.ds(off, 16)] = _f(0.0)
            csv[pl.ds(off, 16)] = _f(-1.0)
            qiv[pl.ds(off, 16)] = _i(0)
            return carry

        lax.fori_loop(0, _CAP // 16, prefill, 0)

        def compact(ck, cnt):
            off = ck * 16
            sc = sv[pl.ds(off, 16)]
            m = sc > CONF_THRESH
            plsc.store_compressed(c1v.at[pl.ds(cnt, 16)], x1v[pl.ds(off, 16)], mask=m)
            plsc.store_compressed(c2v.at[pl.ds(cnt, 16)], y1v[pl.ds(off, 16)], mask=m)
            plsc.store_compressed(c3v.at[pl.ds(cnt, 16)], x2v[pl.ds(off, 16)], mask=m)
            plsc.store_compressed(c4v.at[pl.ds(cnt, 16)], y2v[pl.ds(off, 16)], mask=m)
            plsc.store_compressed(csv.at[pl.ds(cnt, 16)], sc, mask=m)
            plsc.store_compressed(qiv.at[pl.ds(cnt, 16)], _i(base + off) + lane, mask=m)
            return cnt + jnp.max(plsc.all_reduce_population_count(m))

        cnt = lax.fori_loop(0, _CHUNKS, compact, jnp.int32(0))
        nblk = (cnt + 63) // 64

        def fused_pass(wx1, wy1, wx2, wy2):
            area_a = (wx2 - wx1) * (wy2 - wy1)

            def blk_body(blk, carry):
                bv, bi = carry
                for jj in range(4):
                    off = blk * 64 + jj * 16
                    xa = c1v[pl.ds(off, 16)]
                    ya = c2v[pl.ds(off, 16)]
                    xb = c3v[pl.ds(off, 16)]
                    yb = c4v[pl.ds(off, 16)]
                    sc = csv[pl.ds(off, 16)]
                    ix1 = jnp.maximum(_f(wx1), xa)
                    iy1 = jnp.maximum(_f(wy1), ya)
                    ix2 = jnp.minimum(_f(wx2), xb)
                    iy2 = jnp.minimum(_f(wy2), yb)
                    inter = (jnp.maximum(ix2 - ix1, _f(0.0))
                             * jnp.maximum(iy2 - iy1, _f(0.0)))
                    area_b = (xb - xa) * (yb - ya)
                    iou = inter / (_f(area_a) + area_b - inter + _f(1e-9))
                    sc = jnp.where(iou > IOU_THRESH, _f(-1.0), sc)
                    csv[pl.ds(off, 16)] = sc
                    cond = sc > bv
                    bv = jnp.where(cond, sc, bv)
                    bi = jnp.where(cond, _i(off) + lane, bi)
                return bv, bi

            bv, bi = plsc.parallel_loop(
                0, nblk, step=1, unroll=2, carry=(_f(-2.0), _i(2 ** 30)))(blk_body)
            lmax = jnp.max(bv)
            lidx = jnp.min(jnp.where(bv == _f(lmax), bi, _i(2 ** 30)))
            return lmax, lidx

        lmax0, lidx0 = fused_pass(jnp.float32(-10.0), jnp.float32(-10.0),
                                  jnp.float32(-9.0), jnp.float32(-9.0))

        def pick(i, carry):
            lmax, lidx = carry
            loc = _i(jnp.minimum(lidx, _CAP - 1))
            gid = plsc.load_gather(qiv, [loc])
            cx1 = plsc.load_gather(c1v, [loc])
            cy1 = plsc.load_gather(c2v, [loc])
            cx2 = plsc.load_gather(c3v, [loc])
            cy2 = plsc.load_gather(c4v, [loc])
            pub = jnp.where(lane == _i(0), _f(lmax), _f(0.0))
            pub = jnp.where(lane == _i(1), gid.astype(jnp.float32), pub)
            pub = jnp.where(lane == _i(2), cx1, pub)
            pub = jnp.where(lane == _i(3), cy1, pub)
            pub = jnp.where(lane == _i(4), cx2, pub)
            pub = jnp.where(lane == _i(5), cy2, pub)
            pubv[...] = pub
            par = lax.rem(i, 2)
            pltpu.sync_copy(pubv, shared.at[par, t])
            plsc.subcore_barrier()
            pltpu.sync_copy(shared.at[par], rdv)

            def col(j):
                return plsc.load_gather(rdv, [lane, _i(j)])

            scores = col(0)
            idxs = col(1)
            m = jnp.max(scores)
            widx = jnp.min(jnp.where(scores == _f(m), idxs, _f(3.0e7)))
            wsel = (scores == _f(m)) & (idxs == _f(widx))
            wx1 = jnp.max(jnp.where(wsel, col(2), _f(-3.0e7)))
            wy1 = jnp.max(jnp.where(wsel, col(3), _f(-3.0e7)))
            wx2 = jnp.max(jnp.where(wsel, col(4), _f(-3.0e7)))
            wy2 = jnp.max(jnp.where(wsel, col(5), _f(-3.0e7)))
            ok = m > 0.0

            @pl.when(t == 0)
            def _emit():
                row = jnp.where(lane == _i(0), _f(wx1), _f(0.0))
                row = jnp.where(lane == _i(1), _f(wy1), row)
                row = jnp.where(lane == _i(2), _f(wx2), row)
                row = jnp.where(lane == _i(3), _f(wy2), row)
                row = jnp.where(lane == _i(4), _f(m), row)
                row = jnp.where(jnp.full((16,), ok, jnp.bool_), row, _f(0.0))
                plsc.store_scatter(outv, [_i(i), lane], row)

            return fused_pass(wx1, wy1, wx2, wy2)

        lax.fori_loop(0, MAX_OUT, pick, (lmax0, lidx0))

        @pl.when(t == 0)
        def _flush():
            pltpu.sync_copy(outv, out_h)


@jax.jit
def kernel(boxes, scores):
    padn = _NP - N_BOXES
    bt = jnp.pad(boxes, ((0, padn), (0, 0))).T
    sp = jnp.pad(scores, (0, padn), constant_values=-1.0)
    out = _nms_sc(bt[0], bt[1], bt[2], bt[3], sp)
    return out[:, :5]
